# baseline (speedup 1.0000x reference)
import functools

import jax
import jax.numpy as jnp
from jax import lax
from jax.experimental import pallas as pl
from jax.experimental.pallas import tpu as pltpu
from jax.experimental.pallas import tpu_sc as plsc

N_USERS = 50000
N_ITEMS = 50000
N_TOTAL = N_USERS + N_ITEMS
N_PAD = 100352
H = 16
NNZ = 1600000
BATCH = 16384
REG = 1e-4

NT = 16
CH = 128
SUB = 3
SUP = SUB * CH
SC_N = 263
ETILE = SC_N * SUP
NNZ_PAD = NT * ETILE
ECS = NNZ_PAD + SUP
RPT = N_PAD // NT
DCH = 224
SPT = BATCH // NT
SCH = SPT // CH

_mesh = plsc.VectorSubcoreMesh(core_axis_name="c", subcore_axis_name="s")


@functools.partial(
    pl.kernel,
    out_type=(
        jax.ShapeDtypeStruct((2 * N_PAD, H), jnp.float32),
        jax.ShapeDtypeStruct((2 * BATCH, H), jnp.float32),
        jax.ShapeDtypeStruct((2 * BATCH, H), jnp.float32),
        jax.ShapeDtypeStruct((2 * BATCH, H), jnp.float32),
    ),
    mesh=_mesh,
    compiler_params=pltpu.CompilerParams(use_tc_tiling_on_sc=False),
    scratch_types=[
        pltpu.VMEM((SUP,), jnp.int32),
        pltpu.VMEM((SUP,), jnp.int32),
        pltpu.VMEM((SUP,), jnp.int32),
        pltpu.VMEM((SUP,), jnp.int32),
        pltpu.VMEM((SUP,), jnp.int32),
        pltpu.VMEM((SUP,), jnp.int32),
        pltpu.VMEM((SUP,), jnp.float32),
        pltpu.VMEM((SUP,), jnp.float32),
        pltpu.VMEM((SUP,), jnp.float32),
        pltpu.VMEM((SUP, H), jnp.float32),
        pltpu.VMEM((SUP, H), jnp.float32),
        pltpu.VMEM((SUP, H), jnp.float32),
        pltpu.VMEM((DCH, H), jnp.float32),
        pltpu.VMEM((DCH, H), jnp.float32),
        pltpu.SemaphoreType.DMA,
        pltpu.SemaphoreType.DMA,
        pltpu.SemaphoreType.DMA,
        pltpu.SemaphoreType.DMA,
        pltpu.SemaphoreType.DMA,
        pltpu.SemaphoreType.DMA,
        pltpu.SemaphoreType.DMA,
        pltpu.SemaphoreType.DMA,
        pltpu.SemaphoreType.DMA,
        pltpu.VMEM_SHARED((N_PAD, H), jnp.float32),
    ],
)
def _sc_propagate(e0, rows, cols2, vals, uidx2, pidx2, nidx2,
                  e1, uemb, pemb, nemb,
                  colv0, colv1, colv2, rowv0, rowv1, rowv2,
                  valv0, valv1, valv2, gbuf0, gbuf1, gbuf2,
                  bounce, zbuf,
                  esem0, esem1, esem2, gsem0, gsem1, gsem2,
                  ssem0, ssem1, ssem2, acc):
    c = lax.axis_index("c")
    t = lax.axis_index("s")
    cbase = c * N_PAD
    rbase = t * RPT
    ebase0 = t * ETILE

    def zrow(i, _):
        zbuf[i, :] = jnp.zeros((H,), jnp.float32)
        return 0

    lax.fori_loop(0, DCH, zrow, 0)

    for k in range(RPT // DCH):
        pltpu.sync_copy(zbuf, acc.at[pl.ds(rbase + k * DCH, DCH)])
    plsc.subcore_barrier()

    colb = (colv0, colv1, colv2)
    rowb = (rowv0, rowv1, rowv2)
    valb = (valv0, valv1, valv2)
    gb = (gbuf0, gbuf1, gbuf2)
    es = (esem0, esem1, esem2)
    gs = (gsem0, gsem1, gsem2)
    ss = (ssem0, ssem1, ssem2)

    def edge_pass(src):
        def fire_idx(s, b):
            base = ebase0 + s * SUP
            pltpu.async_copy(cols2.at[pl.ds(c * ECS + base, SUP)], colb[b], es[b])
            pltpu.async_copy(rows.at[pl.ds(base, SUP)], rowb[b], es[b])
            pltpu.async_copy(vals.at[pl.ds(base, SUP)], valb[b], es[b])

        def wait_idx(b):
            pltpu.make_async_copy(e0.at[pl.ds(0, 3 * SUP // 16)],
                                  gb[b].at[pl.ds(0, 3 * SUP // 16)], es[b]).wait()

        def fire_gathers(b):
            for k in range(SUB):
                pltpu.async_copy(src.at[colb[b].at[pl.ds(k * CH, CH)]],
                                 gb[b].at[pl.ds(k * CH, CH)], gs[b])

        def wait_gathers(b):
            pltpu.make_async_copy(e0.at[pl.ds(0, SUP)], gb[b], gs[b]).wait()

        def process(b):
            g_ = gb[b]
            v_ = valb[b]

            @functools.partial(plsc.parallel_loop, 0, SUP // 16, unroll=4)
            def scale(jg):
                vrow = v_[pl.ds(jg * 16, 16)]
                for l in range(16):
                    j = jg * 16 + l
                    g_[j, :] = g_[j, :] * vrow[l]
            pltpu.async_copy(g_, acc.at[rowb[b]], ss[b], add=True)

        def wait_scatter(b):
            pltpu.make_async_copy(gb[b], acc.at[rowb[b]], ss[b]).wait()

        fire_idx(0, 0)
        fire_idx(1, 1)
        wait_idx(0)
        fire_gathers(0)
        fire_idx(2, 2)
        wait_idx(1)
        fire_gathers(1)
        wait_gathers(0)
        process(0)

        def body_one(s, b):
            bm = (b + 2) % 3
            bp = (b + 1) % 3
            wait_scatter(bp)
            fire_idx(s + 1, bp)
            wait_idx(b)
            fire_gathers(b)
            wait_gathers(bm)
            process(bm)

        def body(gg, _):
            s0 = 3 * gg + 2
            body_one(s0, 2)
            body_one(s0 + 1, 0)
            body_one(s0 + 2, 1)
            return 0

        lax.fori_loop(0, (SC_N - 2) // 3, body, 0)
        wait_gathers(1)
        process(1)
        wait_idx(2)
        wait_scatter(0)
        wait_scatter(1)

    def dump(dst_hbm, rezero):
        for k in range(RPT // DCH):
            sl = pl.ds(rbase + k * DCH, DCH)
            pltpu.sync_copy(acc.at[sl], bounce)
            pltpu.sync_copy(bounce, dst_hbm.at[pl.ds(cbase + rbase + k * DCH, DCH)])
            if rezero:
                pltpu.sync_copy(zbuf, acc.at[sl])

    edge_pass(e0)
    plsc.subcore_barrier()
    dump(e1, rezero=True)
    plsc.subcore_barrier()
    edge_pass(e1)
    plsc.subcore_barrier()

    sbase = t * SPT
    iv = colv0.at[pl.ds(0, CH)]
    lv = colv1.at[pl.ds(0, CH)]
    d0 = gbuf0.at[pl.ds(0, CH)]
    d1 = gbuf1.at[pl.ds(0, CH)]
    d2 = gbuf2.at[pl.ds(0, CH)]

    def sample_pass(idx2_hbm, dst_hbm):
        def samp_chunk(q, _):
            b0 = sbase + q * CH
            pltpu.sync_copy(idx2_hbm.at[pl.ds(c * BATCH + b0, CH)], iv)
            for k in range(CH // 16):
                sl = pl.ds(k * 16, 16)
                colv1[sl] = colv0[sl] - cbase
            pltpu.async_copy(e0.at[iv], d0, gsem0)
            pltpu.async_copy(e1.at[iv], d1, gsem1)
            pltpu.async_copy(acc.at[lv], d2, gsem2)
            pltpu.make_async_copy(e0.at[iv], d0, gsem0).wait()
            pltpu.make_async_copy(e1.at[iv], d1, gsem1).wait()
            pltpu.make_async_copy(acc.at[lv], d2, gsem2).wait()

            @functools.partial(plsc.parallel_loop, 0, CH // 16, unroll=2)
            def srow(jg):
                for l in range(16):
                    j = jg * 16 + l
                    gbuf0[j, :] = gbuf0[j, :] + gbuf1[j, :] + gbuf2[j, :]
            pltpu.sync_copy(d0, dst_hbm.at[pl.ds(c * BATCH + b0, CH)])
            return 0

        lax.fori_loop(0, SCH, samp_chunk, 0)

    sample_pass(uidx2, uemb)
    sample_pass(pidx2, pemb)
    sample_pass(nidx2, nemb)


def _tc_loss_body(u_ref, p_ref, n_ref, out_ref):
    u = u_ref[...]
    p = p_ref[...]
    n = n_ref[...]
    seg = (lax.broadcasted_iota(jnp.int32, (128, 8), 0) // 16
           == lax.broadcasted_iota(jnp.int32, (128, 8), 1)).astype(jnp.float32)
    mp = u[0] * p[0] + u[1] * p[1]
    mn = u[0] * n[0] + u[1] * n[1]
    pos = jax.lax.dot(mp, seg, precision=jax.lax.Precision.HIGHEST)
    neg = jax.lax.dot(mn, seg, precision=jax.lax.Precision.HIGHEST)
    x = (pos - neg) * (1.0 / 9.0)
    softplus_negx = jnp.maximum(-x, 0.0) + jnp.log1p(jnp.exp(-jnp.abs(x)))
    loss1 = jnp.sum(softplus_negx) / BATCH
    reg_sum = (jnp.sum(u * u) + jnp.sum(p * p) + jnp.sum(n * n)) * (1.0 / 9.0)
    out_ref[0, 0] = loss1 + REG * 0.5 * reg_sum / BATCH


_tc_loss = pl.pallas_call(
    _tc_loss_body,
    out_shape=jax.ShapeDtypeStruct((1, 1), jnp.float32),
    out_specs=pl.BlockSpec(memory_space=pltpu.SMEM),
)


def kernel(users, pos_items, neg_items, user_table, item_table, adj_idx, adj_val):
    all_emb = jnp.concatenate([user_table, item_table], axis=0)
    zpad = jnp.zeros((N_PAD - N_TOTAL, H), jnp.float32)
    e0 = jnp.concatenate([all_emb[:, :H], zpad, all_emb[:, H:], zpad], axis=0)
    rows = adj_idx[0].astype(jnp.int32)
    cols = adj_idx[1].astype(jnp.int32)
    extra = NNZ_PAD - NNZ + SUP
    fill = (jnp.arange(extra, dtype=jnp.int32) * 97) % N_TOTAL
    rows_p = jnp.concatenate([rows, fill])
    cols_p = jnp.concatenate([cols, fill])
    vals_p = jnp.concatenate([adj_val, jnp.zeros((extra,), jnp.float32)])
    cols2 = jnp.concatenate([cols_p, cols_p + N_PAD])
    u32 = users.astype(jnp.int32)
    p32 = pos_items.astype(jnp.int32) + N_USERS
    n32 = neg_items.astype(jnp.int32) + N_USERS
    uidx2 = jnp.concatenate([u32, u32 + N_PAD])
    pidx2 = jnp.concatenate([p32, p32 + N_PAD])
    nidx2 = jnp.concatenate([n32, n32 + N_PAD])
    e1, uemb, pemb, nemb = _sc_propagate(
        e0, rows_p, cols2, vals_p, uidx2, pidx2, nidx2)
    del e1
    loss = _tc_loss(uemb.reshape(2, 2048, 128), pemb.reshape(2, 2048, 128),
                    nemb.reshape(2, 2048, 128))
    return loss.reshape(())


# speedup vs baseline: 1.0154x; 1.0154x over previous
import functools

import jax
import jax.numpy as jnp
from jax import lax
from jax.experimental import pallas as pl
from jax.experimental.pallas import tpu as pltpu
from jax.experimental.pallas import tpu_sc as plsc

N_USERS = 50000
N_ITEMS = 50000
N_TOTAL = N_USERS + N_ITEMS
N_PAD = 100352
H = 16
NNZ = 1600000
BATCH = 16384
REG = 1e-4

NT = 16
CH = 128
SUB = 2
SUP = SUB * CH
SC_N = 395
ETILE = SC_N * SUP
NNZ_PAD = NT * ETILE
ECS = NNZ_PAD + SUP
RPT = N_PAD // NT
DCH = 224
SPT = BATCH // NT
SCH = SPT // CH

_mesh = plsc.VectorSubcoreMesh(core_axis_name="c", subcore_axis_name="s")


@functools.partial(
    pl.kernel,
    out_type=(
        jax.ShapeDtypeStruct((2 * N_PAD, H), jnp.float32),
        jax.ShapeDtypeStruct((2 * BATCH, H), jnp.float32),
        jax.ShapeDtypeStruct((2 * BATCH, H), jnp.float32),
        jax.ShapeDtypeStruct((2 * BATCH, H), jnp.float32),
    ),
    mesh=_mesh,
    compiler_params=pltpu.CompilerParams(use_tc_tiling_on_sc=False),
    scratch_types=[
        pltpu.VMEM((SUP,), jnp.int32),
        pltpu.VMEM((SUP,), jnp.int32),
        pltpu.VMEM((SUP,), jnp.int32),
        pltpu.VMEM((SUP,), jnp.int32),
        pltpu.VMEM((SUP,), jnp.int32),
        pltpu.VMEM((SUP,), jnp.int32),
        pltpu.VMEM((SUP,), jnp.int32),
        pltpu.VMEM((SUP,), jnp.int32),
        pltpu.VMEM((SUP,), jnp.float32),
        pltpu.VMEM((SUP,), jnp.float32),
        pltpu.VMEM((SUP,), jnp.float32),
        pltpu.VMEM((SUP,), jnp.float32),
        pltpu.VMEM((SUP, H), jnp.float32),
        pltpu.VMEM((SUP, H), jnp.float32),
        pltpu.VMEM((SUP, H), jnp.float32),
        pltpu.VMEM((SUP, H), jnp.float32),
        pltpu.VMEM((DCH, H), jnp.float32),
        pltpu.VMEM((DCH, H), jnp.float32),
        pltpu.SemaphoreType.DMA,
        pltpu.SemaphoreType.DMA,
        pltpu.SemaphoreType.DMA,
        pltpu.SemaphoreType.DMA,
        pltpu.SemaphoreType.DMA,
        pltpu.SemaphoreType.DMA,
        pltpu.SemaphoreType.DMA,
        pltpu.SemaphoreType.DMA,
        pltpu.SemaphoreType.DMA,
        pltpu.SemaphoreType.DMA,
        pltpu.SemaphoreType.DMA,
        pltpu.SemaphoreType.DMA,
        pltpu.VMEM_SHARED((N_PAD, H), jnp.float32),
    ],
)
def _sc_propagate(e0, rows, cols2, vals, uidx2, pidx2, nidx2,
                  e1, uemb, pemb, nemb,
                  colv0, colv1, colv2, colv3, rowv0, rowv1, rowv2, rowv3,
                  valv0, valv1, valv2, valv3, gbuf0, gbuf1, gbuf2, gbuf3,
                  bounce, zbuf,
                  esem0, esem1, esem2, esem3, gsem0, gsem1, gsem2, gsem3,
                  ssem0, ssem1, ssem2, ssem3, acc):
    c = lax.axis_index("c")
    t = lax.axis_index("s")
    cbase = c * N_PAD
    rbase = t * RPT
    ebase0 = t * ETILE

    def zrow(i, _):
        zbuf[i, :] = jnp.zeros((H,), jnp.float32)
        return 0

    lax.fori_loop(0, DCH, zrow, 0)

    for k in range(RPT // DCH):
        pltpu.sync_copy(zbuf, acc.at[pl.ds(rbase + k * DCH, DCH)])
    plsc.subcore_barrier()

    colb = (colv0, colv1, colv2, colv3)
    rowb = (rowv0, rowv1, rowv2, rowv3)
    valb = (valv0, valv1, valv2, valv3)
    gb = (gbuf0, gbuf1, gbuf2, gbuf3)
    es = (esem0, esem1, esem2, esem3)
    gs = (gsem0, gsem1, gsem2, gsem3)
    ss = (ssem0, ssem1, ssem2, ssem3)

    def edge_pass(src):
        def fire_idx(s, b):
            base = ebase0 + s * SUP
            pltpu.async_copy(cols2.at[pl.ds(c * ECS + base, SUP)], colb[b], es[b])
            pltpu.async_copy(rows.at[pl.ds(base, SUP)], rowb[b], es[b])
            pltpu.async_copy(vals.at[pl.ds(base, SUP)], valb[b], es[b])

        def wait_idx(b):
            pltpu.make_async_copy(e0.at[pl.ds(0, 3 * SUP // 16)],
                                  gb[b].at[pl.ds(0, 3 * SUP // 16)], es[b]).wait()

        def fire_gathers(b):
            for k in range(SUB):
                pltpu.async_copy(src.at[colb[b].at[pl.ds(k * CH, CH)]],
                                 gb[b].at[pl.ds(k * CH, CH)], gs[b])

        def wait_gathers(b):
            pltpu.make_async_copy(e0.at[pl.ds(0, SUP)], gb[b], gs[b]).wait()

        def process(b):
            g_ = gb[b]
            v_ = valb[b]

            @functools.partial(plsc.parallel_loop, 0, SUP // 16, unroll=4)
            def scale(jg):
                vrow = v_[pl.ds(jg * 16, 16)]
                for l in range(16):
                    j = jg * 16 + l
                    g_[j, :] = g_[j, :] * vrow[l]
            pltpu.async_copy(g_, acc.at[rowb[b]], ss[b], add=True)

        def wait_scatter(b):
            pltpu.make_async_copy(gb[b], acc.at[rowb[b]], ss[b]).wait()

        fire_idx(0, 0)
        fire_idx(1, 1)
        wait_idx(0)
        fire_gathers(0)
        fire_idx(2, 2)
        wait_idx(1)
        fire_gathers(1)
        fire_idx(3, 3)
        wait_gathers(0)
        process(0)
        wait_idx(2)
        fire_gathers(2)

        def body_one(s, b):
            bm2 = (b + 2) % 4
            bp = (b + 1) % 4
            wait_scatter(bp)
            fire_idx(s + 1, bp)
            wait_idx(b)
            fire_gathers(b)
            wait_gathers(bm2)
            process(bm2)

        def body(gg, _):
            s0 = 4 * gg + 3
            body_one(s0, 3)
            body_one(s0 + 1, 0)
            body_one(s0 + 2, 1)
            body_one(s0 + 3, 2)
            return 0

        lax.fori_loop(0, (SC_N - 3) // 4, body, 0)
        wait_gathers((SC_N - 2) % 4)
        process((SC_N - 2) % 4)
        wait_gathers((SC_N - 1) % 4)
        process((SC_N - 1) % 4)
        wait_idx(SC_N % 4)
        wait_scatter((SC_N - 3) % 4)
        wait_scatter((SC_N - 2) % 4)
        wait_scatter((SC_N - 1) % 4)

    def dump(dst_hbm, rezero):
        for k in range(RPT // DCH):
            sl = pl.ds(rbase + k * DCH, DCH)
            pltpu.sync_copy(acc.at[sl], bounce)
            pltpu.sync_copy(bounce, dst_hbm.at[pl.ds(cbase + rbase + k * DCH, DCH)])
            if rezero:
                pltpu.sync_copy(zbuf, acc.at[sl])

    edge_pass(e0)
    plsc.subcore_barrier()
    dump(e1, rezero=True)
    plsc.subcore_barrier()
    edge_pass(e1)
    plsc.subcore_barrier()

    sbase = t * SPT
    iv = colv0.at[pl.ds(0, CH)]
    lv = colv1.at[pl.ds(0, CH)]
    d0 = gbuf0.at[pl.ds(0, CH)]
    d1 = gbuf1.at[pl.ds(0, CH)]
    d2 = gbuf2.at[pl.ds(0, CH)]

    def sample_pass(idx2_hbm, dst_hbm):
        def samp_chunk(q, _):
            b0 = sbase + q * CH
            pltpu.sync_copy(idx2_hbm.at[pl.ds(c * BATCH + b0, CH)], iv)
            for k in range(CH // 16):
                sl = pl.ds(k * 16, 16)
                colv1[sl] = colv0[sl] - cbase
            pltpu.async_copy(e0.at[iv], d0, gsem0)
            pltpu.async_copy(e1.at[iv], d1, gsem1)
            pltpu.async_copy(acc.at[lv], d2, gsem2)
            pltpu.make_async_copy(e0.at[iv], d0, gsem0).wait()
            pltpu.make_async_copy(e1.at[iv], d1, gsem1).wait()
            pltpu.make_async_copy(acc.at[lv], d2, gsem2).wait()

            @functools.partial(plsc.parallel_loop, 0, CH // 16, unroll=2)
            def srow(jg):
                for l in range(16):
                    j = jg * 16 + l
                    gbuf0[j, :] = gbuf0[j, :] + gbuf1[j, :] + gbuf2[j, :]
            pltpu.sync_copy(d0, dst_hbm.at[pl.ds(c * BATCH + b0, CH)])
            return 0

        lax.fori_loop(0, SCH, samp_chunk, 0)

    sample_pass(uidx2, uemb)
    sample_pass(pidx2, pemb)
    sample_pass(nidx2, nemb)


def _tc_loss_body(u_ref, p_ref, n_ref, out_ref):
    u = u_ref[...]
    p = p_ref[...]
    n = n_ref[...]
    seg = (lax.broadcasted_iota(jnp.int32, (128, 8), 0) // 16
           == lax.broadcasted_iota(jnp.int32, (128, 8), 1)).astype(jnp.float32)
    mp = u[0] * p[0] + u[1] * p[1]
    mn = u[0] * n[0] + u[1] * n[1]
    pos = jax.lax.dot(mp, seg, precision=jax.lax.Precision.HIGHEST)
    neg = jax.lax.dot(mn, seg, precision=jax.lax.Precision.HIGHEST)
    x = (pos - neg) * (1.0 / 9.0)
    softplus_negx = jnp.maximum(-x, 0.0) + jnp.log1p(jnp.exp(-jnp.abs(x)))
    loss1 = jnp.sum(softplus_negx) / BATCH
    reg_sum = (jnp.sum(u * u) + jnp.sum(p * p) + jnp.sum(n * n)) * (1.0 / 9.0)
    out_ref[0, 0] = loss1 + REG * 0.5 * reg_sum / BATCH


_tc_loss = pl.pallas_call(
    _tc_loss_body,
    out_shape=jax.ShapeDtypeStruct((1, 1), jnp.float32),
    out_specs=pl.BlockSpec(memory_space=pltpu.SMEM),
)


def kernel(users, pos_items, neg_items, user_table, item_table, adj_idx, adj_val):
    all_emb = jnp.concatenate([user_table, item_table], axis=0)
    zpad = jnp.zeros((N_PAD - N_TOTAL, H), jnp.float32)
    e0 = jnp.concatenate([all_emb[:, :H], zpad, all_emb[:, H:], zpad], axis=0)
    rows = adj_idx[0].astype(jnp.int32)
    cols = adj_idx[1].astype(jnp.int32)
    extra = NNZ_PAD - NNZ + SUP
    fill = (jnp.arange(extra, dtype=jnp.int32) * 97) % N_TOTAL
    rows_p = jnp.concatenate([rows, fill])
    cols_p = jnp.concatenate([cols, fill])
    vals_p = jnp.concatenate([adj_val, jnp.zeros((extra,), jnp.float32)])
    cols2 = jnp.concatenate([cols_p, cols_p + N_PAD])
    u32 = users.astype(jnp.int32)
    p32 = pos_items.astype(jnp.int32) + N_USERS
    n32 = neg_items.astype(jnp.int32) + N_USERS
    uidx2 = jnp.concatenate([u32, u32 + N_PAD])
    pidx2 = jnp.concatenate([p32, p32 + N_PAD])
    nidx2 = jnp.concatenate([n32, n32 + N_PAD])
    e1, uemb, pemb, nemb = _sc_propagate(
        e0, rows_p, cols2, vals_p, uidx2, pidx2, nidx2)
    del e1
    loss = _tc_loss(uemb.reshape(2, 2048, 128), pemb.reshape(2, 2048, 128),
                    nemb.reshape(2, 2048, 128))
    return loss.reshape(())


# speedup vs baseline: 1.0327x; 1.0170x over previous
import functools

import jax
import jax.numpy as jnp
from jax import lax
from jax.experimental import pallas as pl
from jax.experimental.pallas import tpu as pltpu
from jax.experimental.pallas import tpu_sc as plsc

N_USERS = 50000
N_ITEMS = 50000
N_TOTAL = N_USERS + N_ITEMS
N_PAD = 100352
H = 16
NNZ = 1600000
BATCH = 16384
REG = 1e-4

NT = 16
CH = 128
SUB = 2
SUP = SUB * CH
SC_N = 395
ETILE = SC_N * SUP
NNZ_PAD = NT * ETILE
ECS = NNZ_PAD + SUP
RPT = N_PAD // NT
DCH = 224
SPT = BATCH // NT
SCH = SPT // CH

_mesh = plsc.VectorSubcoreMesh(core_axis_name="c", subcore_axis_name="s")


@functools.partial(
    pl.kernel,
    out_type=(
        jax.ShapeDtypeStruct((2 * N_PAD, H), jnp.float32),
        jax.ShapeDtypeStruct((2 * BATCH, H), jnp.float32),
        jax.ShapeDtypeStruct((2 * BATCH, H), jnp.float32),
        jax.ShapeDtypeStruct((2 * BATCH, H), jnp.float32),
    ),
    mesh=_mesh,
    compiler_params=pltpu.CompilerParams(use_tc_tiling_on_sc=False),
    scratch_types=[
        pltpu.VMEM((SUP,), jnp.int32),
        pltpu.VMEM((SUP,), jnp.int32),
        pltpu.VMEM((SUP,), jnp.int32),
        pltpu.VMEM((SUP,), jnp.int32),
        pltpu.VMEM((SUP,), jnp.int32),
        pltpu.VMEM((SUP,), jnp.int32),
        pltpu.VMEM((SUP,), jnp.int32),
        pltpu.VMEM((SUP,), jnp.int32),
        pltpu.VMEM((SUP,), jnp.float32),
        pltpu.VMEM((SUP,), jnp.float32),
        pltpu.VMEM((SUP,), jnp.float32),
        pltpu.VMEM((SUP,), jnp.float32),
        pltpu.VMEM((SUP, H), jnp.float32),
        pltpu.VMEM((SUP, H), jnp.float32),
        pltpu.VMEM((SUP, H), jnp.float32),
        pltpu.VMEM((SUP, H), jnp.float32),
        pltpu.VMEM((DCH, H), jnp.float32),
        pltpu.VMEM((DCH, H), jnp.float32),
        pltpu.SemaphoreType.DMA,
        pltpu.SemaphoreType.DMA,
        pltpu.SemaphoreType.DMA,
        pltpu.SemaphoreType.DMA,
        pltpu.SemaphoreType.DMA,
        pltpu.SemaphoreType.DMA,
        pltpu.SemaphoreType.DMA,
        pltpu.SemaphoreType.DMA,
        pltpu.SemaphoreType.DMA,
        pltpu.SemaphoreType.DMA,
        pltpu.SemaphoreType.DMA,
        pltpu.SemaphoreType.DMA,
        pltpu.VMEM_SHARED((N_PAD, H), jnp.float32),
    ],
)
def _sc_propagate(e0, rows, cols, vals, uidx, pidx, nidx,
                  e1, uemb, pemb, nemb,
                  colv0, colv1, colv2, colv3, rowv0, rowv1, rowv2, rowv3,
                  valv0, valv1, valv2, valv3, gbuf0, gbuf1, gbuf2, gbuf3,
                  bounce, zbuf,
                  esem0, esem1, esem2, esem3, gsem0, gsem1, gsem2, gsem3,
                  ssem0, ssem1, ssem2, ssem3, acc):
    c = lax.axis_index("c")
    t = lax.axis_index("s")
    cbase = c * N_PAD
    rbase = t * RPT
    ebase0 = t * ETILE

    def zrow(i, _):
        zbuf[i, :] = jnp.zeros((H,), jnp.float32)
        return 0

    lax.fori_loop(0, DCH, zrow, 0)

    for k in range(RPT // DCH):
        pltpu.sync_copy(zbuf, acc.at[pl.ds(rbase + k * DCH, DCH)])
    plsc.subcore_barrier()

    colb = (colv0, colv1, colv2, colv3)
    rowb = (rowv0, rowv1, rowv2, rowv3)
    valb = (valv0, valv1, valv2, valv3)
    gb = (gbuf0, gbuf1, gbuf2, gbuf3)
    es = (esem0, esem1, esem2, esem3)
    gs = (gsem0, gsem1, gsem2, gsem3)
    ss = (ssem0, ssem1, ssem2, ssem3)

    def edge_pass(src):
        def fire_idx(s, b):
            base = ebase0 + s * SUP
            pltpu.async_copy(cols.at[pl.ds(base, SUP)], colb[b], es[b])
            pltpu.async_copy(rows.at[pl.ds(base, SUP)], rowb[b], es[b])
            pltpu.async_copy(vals.at[pl.ds(base, SUP)], valb[b], es[b])

        def wait_idx(b):
            pltpu.make_async_copy(e0.at[pl.ds(0, 3 * SUP // 16)],
                                  gb[b].at[pl.ds(0, 3 * SUP // 16)], es[b]).wait()
            for k in range(SUP // 16):
                sl = pl.ds(k * 16, 16)
                colb[b][sl] = colb[b][sl] + cbase

        def fire_gathers(b):
            for k in range(SUB):
                pltpu.async_copy(src.at[colb[b].at[pl.ds(k * CH, CH)]],
                                 gb[b].at[pl.ds(k * CH, CH)], gs[b])

        def wait_gathers(b):
            pltpu.make_async_copy(e0.at[pl.ds(0, SUP)], gb[b], gs[b]).wait()

        def process(b):
            g_ = gb[b]
            v_ = valb[b]

            @functools.partial(plsc.parallel_loop, 0, SUP // 16, unroll=4)
            def scale(jg):
                vrow = v_[pl.ds(jg * 16, 16)]
                for l in range(16):
                    j = jg * 16 + l
                    g_[j, :] = g_[j, :] * vrow[l]
            pltpu.async_copy(g_, acc.at[rowb[b]], ss[b], add=True)

        def wait_scatter(b):
            pltpu.make_async_copy(gb[b], acc.at[rowb[b]], ss[b]).wait()

        fire_idx(0, 0)
        fire_idx(1, 1)
        wait_idx(0)
        fire_gathers(0)
        fire_idx(2, 2)
        wait_idx(1)
        fire_gathers(1)
        fire_idx(3, 3)
        wait_gathers(0)
        process(0)
        wait_idx(2)
        fire_gathers(2)

        def body_one(s, b):
            bm2 = (b + 2) % 4
            bp = (b + 1) % 4
            wait_scatter(bp)
            fire_idx(s + 1, bp)
            wait_idx(b)
            fire_gathers(b)
            wait_gathers(bm2)
            process(bm2)

        def body(gg, _):
            s0 = 4 * gg + 3
            body_one(s0, 3)
            body_one(s0 + 1, 0)
            body_one(s0 + 2, 1)
            body_one(s0 + 3, 2)
            return 0

        lax.fori_loop(0, (SC_N - 3) // 4, body, 0)
        wait_gathers((SC_N - 2) % 4)
        process((SC_N - 2) % 4)
        wait_gathers((SC_N - 1) % 4)
        process((SC_N - 1) % 4)
        wait_idx(SC_N % 4)
        wait_scatter((SC_N - 3) % 4)
        wait_scatter((SC_N - 2) % 4)
        wait_scatter((SC_N - 1) % 4)

    def dump(dst_hbm, rezero):
        for k in range(RPT // DCH):
            sl = pl.ds(rbase + k * DCH, DCH)
            pltpu.sync_copy(acc.at[sl], bounce)
            pltpu.sync_copy(bounce, dst_hbm.at[pl.ds(cbase + rbase + k * DCH, DCH)])
            if rezero:
                pltpu.sync_copy(zbuf, acc.at[sl])

    edge_pass(e0)
    plsc.subcore_barrier()
    dump(e1, rezero=True)
    plsc.subcore_barrier()
    edge_pass(e1)
    plsc.subcore_barrier()

    sbase = t * SPT
    iv = colv0.at[pl.ds(0, CH)]
    lv = colv1.at[pl.ds(0, CH)]
    d0 = gbuf0.at[pl.ds(0, CH)]
    d1 = gbuf1.at[pl.ds(0, CH)]
    d2 = gbuf2.at[pl.ds(0, CH)]

    def sample_pass(idx_hbm, dst_hbm):
        def samp_chunk(q, _):
            b0 = sbase + q * CH
            pltpu.sync_copy(idx_hbm.at[pl.ds(b0, CH)], lv)
            for k in range(CH // 16):
                sl = pl.ds(k * 16, 16)
                colv0[sl] = colv1[sl] + cbase
            pltpu.async_copy(e0.at[iv], d0, gsem0)
            pltpu.async_copy(e1.at[iv], d1, gsem1)
            pltpu.async_copy(acc.at[lv], d2, gsem2)
            pltpu.make_async_copy(e0.at[iv], d0, gsem0).wait()
            pltpu.make_async_copy(e1.at[iv], d1, gsem1).wait()
            pltpu.make_async_copy(acc.at[lv], d2, gsem2).wait()

            @functools.partial(plsc.parallel_loop, 0, CH // 16, unroll=2)
            def srow(jg):
                for l in range(16):
                    j = jg * 16 + l
                    gbuf0[j, :] = gbuf0[j, :] + gbuf1[j, :] + gbuf2[j, :]
            pltpu.sync_copy(d0, dst_hbm.at[pl.ds(c * BATCH + b0, CH)])
            return 0

        lax.fori_loop(0, SCH, samp_chunk, 0)

    sample_pass(uidx, uemb)
    sample_pass(pidx, pemb)
    sample_pass(nidx, nemb)


def _tc_loss_body(u_ref, p_ref, n_ref, out_ref):
    u = u_ref[...]
    p = p_ref[...]
    n = n_ref[...]
    seg = (lax.broadcasted_iota(jnp.int32, (128, 8), 0) // 16
           == lax.broadcasted_iota(jnp.int32, (128, 8), 1)).astype(jnp.float32)
    mp = u[0] * p[0] + u[1] * p[1]
    mn = u[0] * n[0] + u[1] * n[1]
    pos = jax.lax.dot(mp, seg, precision=jax.lax.Precision.HIGHEST)
    neg = jax.lax.dot(mn, seg, precision=jax.lax.Precision.HIGHEST)
    x = (pos - neg) * (1.0 / 9.0)
    softplus_negx = jnp.maximum(-x, 0.0) + jnp.log1p(jnp.exp(-jnp.abs(x)))
    loss1 = jnp.sum(softplus_negx) / BATCH
    reg_sum = (jnp.sum(u * u) + jnp.sum(p * p) + jnp.sum(n * n)) * (1.0 / 9.0)
    out_ref[0, 0] = loss1 + REG * 0.5 * reg_sum / BATCH


_tc_loss = pl.pallas_call(
    _tc_loss_body,
    out_shape=jax.ShapeDtypeStruct((1, 1), jnp.float32),
    out_specs=pl.BlockSpec(memory_space=pltpu.SMEM),
)


def kernel(users, pos_items, neg_items, user_table, item_table, adj_idx, adj_val):
    zpad = jnp.zeros((N_PAD - N_TOTAL, H), jnp.float32)
    e0 = jnp.concatenate([user_table[:, :H], item_table[:, :H], zpad,
                          user_table[:, H:], item_table[:, H:], zpad], axis=0)
    rows = adj_idx[0].astype(jnp.int32)
    cols = adj_idx[1].astype(jnp.int32)
    extra = NNZ_PAD - NNZ + SUP
    fill = (jnp.arange(extra, dtype=jnp.int32) * 97) % N_TOTAL
    rows_p = jnp.concatenate([rows, fill])
    cols_p = jnp.concatenate([cols, fill])
    vals_p = jnp.concatenate([adj_val, jnp.zeros((extra,), jnp.float32)])
    u32 = users.astype(jnp.int32)
    p32 = pos_items.astype(jnp.int32) + N_USERS
    n32 = neg_items.astype(jnp.int32) + N_USERS
    e1, uemb, pemb, nemb = _sc_propagate(
        e0, rows_p, cols_p, vals_p, u32, p32, n32)
    del e1
    loss = _tc_loss(uemb.reshape(2, 2048, 128), pemb.reshape(2, 2048, 128),
                    nemb.reshape(2, 2048, 128))
    return loss.reshape(())


# speedup vs baseline: 1.0330x; 1.0004x over previous
import functools

import jax
import jax.numpy as jnp
from jax import lax
from jax.experimental import pallas as pl
from jax.experimental.pallas import tpu as pltpu
from jax.experimental.pallas import tpu_sc as plsc

N_USERS = 50000
N_ITEMS = 50000
N_TOTAL = N_USERS + N_ITEMS
N_PAD = 100352
H = 16
NNZ = 1600000
BATCH = 16384
REG = 1e-4

NT = 16
CH = 128
SUB = 2
SUP = SUB * CH
SC_N = 395
ETILE = SC_N * SUP
NNZ_PAD = NT * ETILE
RPT = N_PAD // NT
DCH = 224
SPT = BATCH // NT
SCH = SPT // CH

_mesh = plsc.VectorSubcoreMesh(core_axis_name="c", subcore_axis_name="s")


@functools.partial(
    pl.kernel,
    out_type=(
        jax.ShapeDtypeStruct((2 * N_PAD, H), jnp.float32),
        jax.ShapeDtypeStruct((2 * BATCH, H), jnp.float32),
        jax.ShapeDtypeStruct((2 * BATCH, H), jnp.float32),
        jax.ShapeDtypeStruct((2 * BATCH, H), jnp.float32),
    ),
    mesh=_mesh,
    compiler_params=pltpu.CompilerParams(use_tc_tiling_on_sc=False),
    scratch_types=[
        pltpu.VMEM((SUP,), jnp.int32),
        pltpu.VMEM((SUP,), jnp.int32),
        pltpu.VMEM((SUP,), jnp.int32),
        pltpu.VMEM((SUP,), jnp.int32),
        pltpu.VMEM((SUP,), jnp.int32),
        pltpu.VMEM((SUP,), jnp.int32),
        pltpu.VMEM((SUP,), jnp.int32),
        pltpu.VMEM((SUP,), jnp.int32),
        pltpu.VMEM((SUP,), jnp.float32),
        pltpu.VMEM((SUP,), jnp.float32),
        pltpu.VMEM((SUP,), jnp.float32),
        pltpu.VMEM((SUP,), jnp.float32),
        pltpu.VMEM((SUP, H), jnp.float32),
        pltpu.VMEM((SUP, H), jnp.float32),
        pltpu.VMEM((SUP, H), jnp.float32),
        pltpu.VMEM((SUP, H), jnp.float32),
        pltpu.VMEM((DCH, H), jnp.float32),
        pltpu.VMEM((DCH, H), jnp.float32),
        pltpu.SemaphoreType.DMA,
        pltpu.SemaphoreType.DMA,
        pltpu.SemaphoreType.DMA,
        pltpu.SemaphoreType.DMA,
        pltpu.SemaphoreType.DMA,
        pltpu.SemaphoreType.DMA,
        pltpu.SemaphoreType.DMA,
        pltpu.SemaphoreType.DMA,
        pltpu.SemaphoreType.DMA,
        pltpu.SemaphoreType.DMA,
        pltpu.SemaphoreType.DMA,
        pltpu.SemaphoreType.DMA,
        pltpu.VMEM_SHARED((N_PAD, H), jnp.float32),
    ],
)
def _sc_propagate(e0, rows, cols, vals, uidx, pidx, nidx,
                  e1, uemb, pemb, nemb,
                  colv0, colv1, colv2, colv3, rowv0, rowv1, rowv2, rowv3,
                  valv0, valv1, valv2, valv3, gbuf0, gbuf1, gbuf2, gbuf3,
                  bounce, zbuf,
                  esem0, esem1, esem2, esem3, gsem0, gsem1, gsem2, gsem3,
                  ssem0, ssem1, ssem2, ssem3, acc):
    c = lax.axis_index("c")
    t = lax.axis_index("s")
    cbase = c * N_PAD
    rbase = t * RPT
    ebase0 = t * ETILE

    def zrow(i, _):
        zbuf[i, :] = jnp.zeros((H,), jnp.float32)
        return 0

    lax.fori_loop(0, DCH, zrow, 0)

    for k in range(RPT // DCH):
        pltpu.sync_copy(zbuf, acc.at[pl.ds(rbase + k * DCH, DCH)])
    plsc.subcore_barrier()

    colb = (colv0, colv1, colv2, colv3)
    rowb = (rowv0, rowv1, rowv2, rowv3)
    valb = (valv0, valv1, valv2, valv3)
    gb = (gbuf0, gbuf1, gbuf2, gbuf3)
    es = (esem0, esem1, esem2, esem3)
    gs = (gsem0, gsem1, gsem2, gsem3)
    ss = (ssem0, ssem1, ssem2, ssem3)

    def edge_pass(src):
        def fire_idx(s, b):
            base = ebase0 + s * SUP
            pltpu.async_copy(cols.at[pl.ds(base, SUP)], colb[b], es[b])
            pltpu.async_copy(rows.at[pl.ds(base, SUP)], rowb[b], es[b])
            pltpu.async_copy(vals.at[pl.ds(base, SUP)], valb[b], es[b])

        def wait_idx(b):
            pltpu.make_async_copy(e0.at[pl.ds(0, 3 * SUP // 16)],
                                  gb[b].at[pl.ds(0, 3 * SUP // 16)], es[b]).wait()
            for k in range(SUP // 16):
                sl = pl.ds(k * 16, 16)
                colb[b][sl] = colb[b][sl] + cbase

        def fire_gathers(b):
            for k in range(SUB):
                pltpu.async_copy(src.at[colb[b].at[pl.ds(k * CH, CH)]],
                                 gb[b].at[pl.ds(k * CH, CH)], gs[b])

        def wait_gathers(b):
            pltpu.make_async_copy(e0.at[pl.ds(0, SUP)], gb[b], gs[b]).wait()

        def process(b):
            g_ = gb[b]
            v_ = valb[b]

            @functools.partial(plsc.parallel_loop, 0, SUP // 16, unroll=4)
            def scale(jg):
                vrow = v_[pl.ds(jg * 16, 16)]
                for l in range(16):
                    j = jg * 16 + l
                    g_[j, :] = g_[j, :] * vrow[l]
            pltpu.async_copy(g_, acc.at[rowb[b]], ss[b], add=True)

        def wait_scatter(b):
            pltpu.make_async_copy(gb[b], acc.at[rowb[b]], ss[b]).wait()

        fire_idx(0, 0)
        fire_idx(1, 1)
        wait_idx(0)
        fire_gathers(0)
        fire_idx(2, 2)
        wait_idx(1)
        fire_gathers(1)
        fire_idx(3, 3)
        wait_gathers(0)
        process(0)
        wait_idx(2)
        fire_gathers(2)

        def body_one(s, b):
            bm2 = (b + 2) % 4
            bp = (b + 1) % 4
            wait_scatter(bp)
            fire_idx(s + 1, bp)
            wait_idx(b)
            fire_gathers(b)
            wait_gathers(bm2)
            process(bm2)

        def body(gg, _):
            s0 = 4 * gg + 3
            body_one(s0, 3)
            body_one(s0 + 1, 0)
            body_one(s0 + 2, 1)
            body_one(s0 + 3, 2)
            return 0

        lax.fori_loop(0, (SC_N - 3) // 4, body, 0)
        wait_gathers((SC_N - 2) % 4)
        process((SC_N - 2) % 4)
        wait_gathers((SC_N - 1) % 4)
        process((SC_N - 1) % 4)
        wait_idx(SC_N % 4)
        wait_scatter((SC_N - 3) % 4)
        wait_scatter((SC_N - 2) % 4)
        wait_scatter((SC_N - 1) % 4)

    def dump(dst_hbm, rezero):
        for k in range(RPT // DCH):
            sl = pl.ds(rbase + k * DCH, DCH)
            pltpu.sync_copy(acc.at[sl], bounce)
            pltpu.sync_copy(bounce, dst_hbm.at[pl.ds(cbase + rbase + k * DCH, DCH)])
            if rezero:
                pltpu.sync_copy(zbuf, acc.at[sl])

    edge_pass(e0)
    plsc.subcore_barrier()
    dump(e1, rezero=True)
    plsc.subcore_barrier()
    edge_pass(e1)
    plsc.subcore_barrier()

    sbase = t * SPT
    iv = colv0.at[pl.ds(0, CH)]
    lv = colv1.at[pl.ds(0, CH)]
    d0 = gbuf0.at[pl.ds(0, CH)]
    d1 = gbuf1.at[pl.ds(0, CH)]
    d2 = gbuf2.at[pl.ds(0, CH)]

    def sample_pass(idx_hbm, dst_hbm):
        def samp_chunk(q, _):
            b0 = sbase + q * CH
            pltpu.sync_copy(idx_hbm.at[pl.ds(b0, CH)], lv)
            for k in range(CH // 16):
                sl = pl.ds(k * 16, 16)
                colv0[sl] = colv1[sl] + cbase
            pltpu.async_copy(e0.at[iv], d0, gsem0)
            pltpu.async_copy(e1.at[iv], d1, gsem1)
            pltpu.async_copy(acc.at[lv], d2, gsem2)
            pltpu.make_async_copy(e0.at[iv], d0, gsem0).wait()
            pltpu.make_async_copy(e1.at[iv], d1, gsem1).wait()
            pltpu.make_async_copy(acc.at[lv], d2, gsem2).wait()

            @functools.partial(plsc.parallel_loop, 0, CH // 16, unroll=2)
            def srow(jg):
                for l in range(16):
                    j = jg * 16 + l
                    gbuf0[j, :] = gbuf0[j, :] + gbuf1[j, :] + gbuf2[j, :]
            pltpu.sync_copy(d0, dst_hbm.at[pl.ds(c * BATCH + b0, CH)])
            return 0

        lax.fori_loop(0, SCH, samp_chunk, 0)

    sample_pass(uidx, uemb)
    sample_pass(pidx, pemb)
    sample_pass(nidx, nemb)


def _tc_loss_body(u_ref, p_ref, n_ref, out_ref):
    u = u_ref[...]
    p = p_ref[...]
    n = n_ref[...]
    seg = (lax.broadcasted_iota(jnp.int32, (128, 8), 0) // 16
           == lax.broadcasted_iota(jnp.int32, (128, 8), 1)).astype(jnp.float32)
    mp = u[0] * p[0] + u[1] * p[1]
    mn = u[0] * n[0] + u[1] * n[1]
    pos = jax.lax.dot(mp, seg, precision=jax.lax.Precision.HIGHEST)
    neg = jax.lax.dot(mn, seg, precision=jax.lax.Precision.HIGHEST)
    x = (pos - neg) * (1.0 / 9.0)
    softplus_negx = jnp.maximum(-x, 0.0) + jnp.log1p(jnp.exp(-jnp.abs(x)))
    loss1 = jnp.sum(softplus_negx) / BATCH
    reg_sum = (jnp.sum(u * u) + jnp.sum(p * p) + jnp.sum(n * n)) * (1.0 / 9.0)
    out_ref[0, 0] = loss1 + REG * 0.5 * reg_sum / BATCH


_tc_loss = pl.pallas_call(
    _tc_loss_body,
    out_shape=jax.ShapeDtypeStruct((1, 1), jnp.float32),
    out_specs=pl.BlockSpec(memory_space=pltpu.SMEM),
)


def kernel(users, pos_items, neg_items, user_table, item_table, adj_idx, adj_val):
    zpad = jnp.zeros((N_PAD - N_TOTAL, H), jnp.float32)
    e0 = jnp.concatenate([user_table[:, :H], item_table[:, :H], zpad,
                          user_table[:, H:], item_table[:, H:], zpad], axis=0)
    rows = adj_idx[0].astype(jnp.int32)
    cols = adj_idx[1].astype(jnp.int32)
    extra = NNZ_PAD - NNZ + SUP
    fill = (jnp.arange(extra, dtype=jnp.int32) * 97) % N_TOTAL
    rows_p = jnp.concatenate([rows, fill])
    cols_p = jnp.concatenate([cols, fill])
    vals_p = jnp.concatenate([adj_val, jnp.zeros((extra,), jnp.float32)])
    u32 = users.astype(jnp.int32)
    p32 = pos_items.astype(jnp.int32) + N_USERS
    n32 = neg_items.astype(jnp.int32) + N_USERS
    e1, uemb, pemb, nemb = _sc_propagate(
        e0, rows_p, cols_p, vals_p, u32, p32, n32)
    del e1
    loss = _tc_loss(uemb.reshape(2, 2048, 128), pemb.reshape(2, 2048, 128),
                    nemb.reshape(2, 2048, 128))
    return loss.reshape(())
